# Initial kernel scaffold; baseline (speedup 1.0000x reference)
import jax
import jax.numpy as jnp
from jax.experimental import pallas as pl


def kernel(nodes, batch, edges, batch_edges, graph_globals, W1, b1, W2, b2):
    raise NotImplementedError("write your pallas kernel here")


# speedup vs baseline: 4.9717x; 4.9717x over previous
import jax
import jax.numpy as jnp
from jax import lax
from jax.experimental import pallas as pl
from jax.experimental.pallas import tpu as pltpu
from jax.experimental.pallas import tpu_sc as plsc

N_GRAPHS = 512
N_NODES = 100000
N_EDGES = 1600000
NODE_DIM = 128
EDGE_DIM = 16
HIDDEN = 64
PACK = NODE_DIM // EDGE_DIM

NW = 32
L = 128

N_FULL = N_NODES // L
N_TAIL = N_NODES - N_FULL * L
N_ITER = (N_FULL + NW - 1) // NW

E_PACKED = N_EDGES // PACK
E_FULL = E_PACKED // L
E_TAIL = E_PACKED - E_FULL * L
E_ITER = (E_FULL + NW - 1) // NW

N_STRAD = 512


def _segsum_body(nodes_hbm, nid_hbm, ntail_id_hbm, epack_hbm, pid_hbm,
                 ptail_id_hbm, np_out, ep_out,
                 nrows, nidx, ntrows, ntidx, erows, eidx, etrows, etidx, zrow,
                 nacc, eacc):
    c = lax.axis_index("c")
    s = lax.axis_index("s")
    wid = c * 16 + s

    def _zero_row(i, _):
        zrow[pl.ds(i * 16, 16)] = jnp.zeros((16,), jnp.float32)
        return _
    lax.fori_loop(0, 8, _zero_row, None)
    base = s * (N_GRAPHS // 16)

    def _zero_nacc(i, _):
        pltpu.sync_copy(zrow, nacc.at[base + i])
        return _
    lax.fori_loop(0, N_GRAPHS // 16, _zero_nacc, None)

    def _zero_eacc(i, _):
        pltpu.sync_copy(zrow, eacc.at[base + i])
        return _
    lax.fori_loop(0, N_GRAPHS // 16, _zero_eacc, None)

    @pl.when(s == 15)
    def _():
        pltpu.sync_copy(zrow, eacc.at[N_GRAPHS])

    plsc.subcore_barrier()

    def _node_chunk(t, _):
        k = wid + t * NW

        @pl.when(k < N_FULL)
        def _():
            pltpu.sync_copy(nodes_hbm.at[pl.ds(k * L, L), :], nrows)
            pltpu.sync_copy(nid_hbm.at[k], nidx.at[0])
            pltpu.sync_copy(nrows, nacc.at[nidx.at[0]], add=True)
        return _
    lax.fori_loop(0, N_ITER, _node_chunk, None)

    @pl.when(wid == 30)
    def _():
        pltpu.sync_copy(nodes_hbm.at[pl.ds(N_FULL * L, N_TAIL), :], ntrows)
        pltpu.sync_copy(ntail_id_hbm, ntidx)
        pltpu.sync_copy(ntrows, nacc.at[ntidx], add=True)

    def _edge_chunk(t, _):
        k = wid + t * NW

        @pl.when(k < E_FULL)
        def _():
            pltpu.sync_copy(epack_hbm.at[pl.ds(k * L, L), :], erows)
            pltpu.sync_copy(pid_hbm.at[k], eidx.at[0])
            pltpu.sync_copy(erows, eacc.at[eidx.at[0]], add=True)
        return _
    lax.fori_loop(0, E_ITER, _edge_chunk, None)

    @pl.when(wid == 31)
    def _():
        pltpu.sync_copy(epack_hbm.at[pl.ds(E_FULL * L, E_TAIL), :], etrows)
        pltpu.sync_copy(ptail_id_hbm, etidx)
        pltpu.sync_copy(etrows, eacc.at[etidx], add=True)

    plsc.subcore_barrier()

    rows = N_GRAPHS // 16
    pltpu.sync_copy(nacc.at[pl.ds(s * rows, rows), :],
                    np_out.at[c, pl.ds(s * rows, rows), :])
    pltpu.sync_copy(eacc.at[pl.ds(s * rows, rows), :],
                    ep_out.at[c, pl.ds(s * rows, rows), :])


def _mlp_body(np_ref, ep_ref, srows_ref, sidsT_ref, gg_ref, w1a_ref, w1b_ref,
              w1c_ref, b1_ref, w2_ref, b2_ref, out_ref):
    ns = np_ref[0] + np_ref[1]
    ep = ep_ref[0] + ep_ref[1]

    gids = lax.broadcasted_iota(jnp.int32, (N_GRAPHS, N_STRAD), 0)
    lane_slot = lax.broadcasted_iota(jnp.int32, (1, NODE_DIM), 1) // EDGE_DIM
    srows = srows_ref[...]
    sidsT = sidsT_ref[...]
    for j in range(PACK):
        oh = (gids == sidsT[j][None, :]).astype(jnp.float32)
        masked = srows * (lane_slot == j).astype(jnp.float32)
        ep = ep + jnp.dot(oh, masked, preferred_element_type=jnp.float32)

    x = (jnp.dot(gg_ref[...], w1a_ref[...], preferred_element_type=jnp.float32)
         + jnp.dot(ns, w1b_ref[...], preferred_element_type=jnp.float32)
         + jnp.dot(ep, w1c_ref[...], preferred_element_type=jnp.float32)
         + b1_ref[...])
    h = jnp.maximum(x, 0.0)
    out_ref[...] = (jnp.dot(h, w2_ref[...], preferred_element_type=jnp.float32)
                    + b2_ref[...])


def kernel(nodes, batch, edges, batch_edges, graph_globals, W1, b1, W2, b2):
    bid = batch.astype(jnp.int32)
    eid = batch_edges.astype(jnp.int32)
    nid2d = lax.slice(bid, (0,), (N_FULL * L,)).reshape(N_FULL, L)
    ntail_id = lax.slice(bid, (N_FULL * L,), (N_NODES,))

    epack = edges.reshape(E_PACKED, NODE_DIM)
    eids8 = eid.reshape(E_PACKED, PACK)
    uniform = jnp.all(eids8 == eids8[:, :1], axis=1)
    pid = jnp.where(uniform, eids8[:, 0], N_GRAPHS)
    pid2d = lax.slice(pid, (0,), (E_FULL * L,)).reshape(E_FULL, L)
    ptail_id = lax.slice(pid, (E_FULL * L,), (E_PACKED,))

    (sidx,) = jnp.nonzero(~uniform, size=N_STRAD, fill_value=0)
    valid = jnp.arange(N_STRAD) < jnp.sum(~uniform)
    srows = jnp.where(valid[:, None], epack[sidx], 0.0)
    sidsT = jnp.where(valid[None, :], eids8[sidx].T, N_GRAPHS).astype(jnp.int32)

    mesh = plsc.VectorSubcoreMesh(core_axis_name="c", subcore_axis_name="s")
    segsum = pl.kernel(
        _segsum_body,
        out_type=[
            jax.ShapeDtypeStruct((2, N_GRAPHS, NODE_DIM), jnp.float32),
            jax.ShapeDtypeStruct((2, N_GRAPHS, NODE_DIM), jnp.float32),
        ],
        mesh=mesh,
        scratch_types=[
            pltpu.VMEM((L, NODE_DIM), jnp.float32),
            pltpu.VMEM((1, L), jnp.int32),
            pltpu.VMEM((N_TAIL, NODE_DIM), jnp.float32),
            pltpu.VMEM((N_TAIL,), jnp.int32),
            pltpu.VMEM((L, NODE_DIM), jnp.float32),
            pltpu.VMEM((1, L), jnp.int32),
            pltpu.VMEM((E_TAIL, NODE_DIM), jnp.float32),
            pltpu.VMEM((E_TAIL,), jnp.int32),
            pltpu.VMEM((NODE_DIM,), jnp.float32),
            pltpu.VMEM_SHARED((N_GRAPHS, NODE_DIM), jnp.float32),
            pltpu.VMEM_SHARED((N_GRAPHS + 1, NODE_DIM), jnp.float32),
        ],
    )
    np_part, ep_part = segsum(nodes, nid2d, ntail_id, epack, pid2d, ptail_id)

    w1a = lax.slice(W1, (0, 0), (NODE_DIM, HIDDEN))
    w1b = lax.slice(W1, (NODE_DIM, 0), (2 * NODE_DIM, HIDDEN))
    w1c = lax.slice(W1, (2 * NODE_DIM, 0), (2 * NODE_DIM + EDGE_DIM, HIDDEN))
    w1c_big = jnp.tile(w1c, (PACK, 1))

    out = pl.pallas_call(
        _mlp_body,
        out_shape=jax.ShapeDtypeStruct((N_GRAPHS, NODE_DIM), jnp.float32),
    )(np_part, ep_part, srows, sidsT, graph_globals, w1a, w1b, w1c_big,
      b1.reshape(1, HIDDEN), W2, b2.reshape(1, NODE_DIM))
    return out


# speedup vs baseline: 5.3597x; 1.0780x over previous
import jax
import jax.numpy as jnp
from jax import lax
from jax.experimental import pallas as pl
from jax.experimental.pallas import tpu as pltpu
from jax.experimental.pallas import tpu_sc as plsc

N_GRAPHS = 512
N_NODES = 100000
N_EDGES = 1600000
NODE_DIM = 128
EDGE_DIM = 16
HIDDEN = 64
PACK = NODE_DIM // EDGE_DIM

NW = 32
L = 128

N_FULL = N_NODES // L
N_TAIL = N_NODES - N_FULL * L
N_CNT = N_FULL // NW
N_EXTRA = N_FULL - N_CNT * NW
N_MAX = N_CNT + 1

E_PACKED = N_EDGES // PACK
E_FULL = E_PACKED // L
E_TAIL = E_PACKED - E_FULL * L
E_CNT = E_FULL // NW
E_EXTRA = E_FULL - E_CNT * NW
E_MAX = E_CNT + 1

N_STRAD = 512


def _phase(src_hbm, idxb, acc, buf0, buf1, sem0, sem1, start, cnt, iter_max,
           idx_off):
    @pl.when(cnt > 0)
    def _():
        pltpu.async_copy(src_hbm.at[pl.ds(start * L, L), :], buf0, sem0)

    @pl.when(cnt > 1)
    def _():
        pltpu.async_copy(src_hbm.at[pl.ds((start + 1) * L, L), :], buf1, sem1)

    def _pair(tp, _):
        t0 = 2 * tp
        t1 = t0 + 1

        @pl.when(t0 < cnt)
        def _():
            pltpu.make_async_copy(
                src_hbm.at[pl.ds((start + t0) * L, L), :], buf0, sem0).wait()
            pltpu.sync_copy(buf0, acc.at[idxb.at[idx_off + t0]], add=True)

            @pl.when(t0 + 2 < cnt)
            def _():
                pltpu.async_copy(
                    src_hbm.at[pl.ds((start + t0 + 2) * L, L), :], buf0, sem0)

        @pl.when(t1 < cnt)
        def _():
            pltpu.make_async_copy(
                src_hbm.at[pl.ds((start + t1) * L, L), :], buf1, sem1).wait()
            pltpu.sync_copy(buf1, acc.at[idxb.at[idx_off + t1]], add=True)

            @pl.when(t1 + 2 < cnt)
            def _():
                pltpu.async_copy(
                    src_hbm.at[pl.ds((start + t1 + 2) * L, L), :], buf1, sem1)
        return _

    lax.fori_loop(0, (iter_max + 1) // 2, _pair, None)


def _segsum_body(nodes_hbm, nid_hbm, ntail_id_hbm, epack_hbm, pid_hbm,
                 ptail_id_hbm, np_out, ep_out,
                 buf0, buf1, nidxb, eidxb, ntrows, ntidx, etrows, etidx, zrow,
                 sem0, sem1, nacc, eacc):
    c = lax.axis_index("c")
    s = lax.axis_index("s")
    wid = c * 16 + s

    n_start = wid * N_CNT + jnp.minimum(wid, N_EXTRA)
    n_cnt = N_CNT + jnp.where(wid < N_EXTRA, 1, 0)
    e_start = wid * E_CNT + jnp.minimum(wid, E_EXTRA)
    e_cnt = E_CNT + jnp.where(wid < E_EXTRA, 1, 0)

    n_start8 = pl.multiple_of((n_start // 8) * 8, 8)
    e_start8 = pl.multiple_of((e_start // 8) * 8, 8)
    pltpu.sync_copy(nid_hbm.at[pl.ds(n_start8, 40), :], nidxb)
    pltpu.sync_copy(pid_hbm.at[pl.ds(e_start8, 64), :], eidxb)

    def _zero_row(i, _):
        zrow[pl.ds(i * 16, 16)] = jnp.zeros((16,), jnp.float32)
        return _
    lax.fori_loop(0, 8, _zero_row, None)
    base = s * (N_GRAPHS // 16)

    def _zero_nacc(i, _):
        pltpu.sync_copy(zrow, nacc.at[base + i])
        return _
    lax.fori_loop(0, N_GRAPHS // 16, _zero_nacc, None)

    def _zero_eacc(i, _):
        pltpu.sync_copy(zrow, eacc.at[base + i])
        return _
    lax.fori_loop(0, N_GRAPHS // 16, _zero_eacc, None)

    @pl.when(s == 15)
    def _():
        pltpu.sync_copy(zrow, eacc.at[N_GRAPHS])

    plsc.subcore_barrier()

    _phase(nodes_hbm, nidxb, nacc, buf0, buf1, sem0, sem1,
           n_start, n_cnt, N_MAX, n_start - n_start8)

    @pl.when(wid == 30)
    def _():
        pltpu.sync_copy(nodes_hbm.at[pl.ds(N_FULL * L, N_TAIL), :], ntrows)
        pltpu.sync_copy(ntail_id_hbm, ntidx)
        pltpu.sync_copy(ntrows, nacc.at[ntidx], add=True)

    _phase(epack_hbm, eidxb, eacc, buf0, buf1, sem0, sem1,
           e_start, e_cnt, E_MAX, e_start - e_start8)

    @pl.when(wid == 31)
    def _():
        pltpu.sync_copy(epack_hbm.at[pl.ds(E_FULL * L, E_TAIL), :], etrows)
        pltpu.sync_copy(ptail_id_hbm, etidx)
        pltpu.sync_copy(etrows, eacc.at[etidx], add=True)

    plsc.subcore_barrier()

    rows = N_GRAPHS // 16
    pltpu.sync_copy(nacc.at[pl.ds(s * rows, rows), :],
                    np_out.at[c, pl.ds(s * rows, rows), :])
    pltpu.sync_copy(eacc.at[pl.ds(s * rows, rows), :],
                    ep_out.at[c, pl.ds(s * rows, rows), :])


def _mlp_body(np_ref, ep_ref, srows_ref, sidsT_ref, gg_ref, w1a_ref, w1b_ref,
              w1c_ref, b1_ref, w2_ref, b2_ref, out_ref):
    ns = np_ref[0] + np_ref[1]
    ep = ep_ref[0] + ep_ref[1]

    gids = lax.broadcasted_iota(jnp.int32, (N_GRAPHS, N_STRAD), 0)
    lane_slot = lax.broadcasted_iota(jnp.int32, (1, NODE_DIM), 1) // EDGE_DIM
    srows = srows_ref[...]
    sidsT = sidsT_ref[...]
    for j in range(PACK):
        oh = (gids == sidsT[j][None, :]).astype(jnp.float32)
        masked = srows * (lane_slot == j).astype(jnp.float32)
        ep = ep + jnp.dot(oh, masked, preferred_element_type=jnp.float32)

    x = (jnp.dot(gg_ref[...], w1a_ref[...], preferred_element_type=jnp.float32)
         + jnp.dot(ns, w1b_ref[...], preferred_element_type=jnp.float32)
         + jnp.dot(ep, w1c_ref[...], preferred_element_type=jnp.float32)
         + b1_ref[...])
    h = jnp.maximum(x, 0.0)
    out_ref[...] = (jnp.dot(h, w2_ref[...], preferred_element_type=jnp.float32)
                    + b2_ref[...])


def kernel(nodes, batch, edges, batch_edges, graph_globals, W1, b1, W2, b2):
    bid = batch.astype(jnp.int32)
    eid = batch_edges.astype(jnp.int32)
    nid2d = lax.slice(bid, (0,), (N_FULL * L,)).reshape(N_FULL, L)
    nid2d = jnp.pad(nid2d, ((0, N_MAX + NW), (0, 0)))
    ntail_id = lax.slice(bid, (N_FULL * L,), (N_NODES,))

    epack = edges.reshape(E_PACKED, NODE_DIM)
    eids8 = eid.reshape(E_PACKED, PACK)
    uniform = jnp.all(eids8 == eids8[:, :1], axis=1)
    pid = jnp.where(uniform, eids8[:, 0], N_GRAPHS)
    pid2d = lax.slice(pid, (0,), (E_FULL * L,)).reshape(E_FULL, L)
    pid2d = jnp.pad(pid2d, ((0, E_MAX + NW), (0, 0)))
    ptail_id = lax.slice(pid, (E_FULL * L,), (E_PACKED,))

    (sidx,) = jnp.nonzero(~uniform, size=N_STRAD, fill_value=0)
    valid = jnp.arange(N_STRAD) < jnp.sum(~uniform)
    srows = jnp.where(valid[:, None], epack[sidx], 0.0)
    sidsT = jnp.where(valid[None, :], eids8[sidx].T, N_GRAPHS).astype(jnp.int32)

    mesh = plsc.VectorSubcoreMesh(core_axis_name="c", subcore_axis_name="s")
    segsum = pl.kernel(
        _segsum_body,
        out_type=[
            jax.ShapeDtypeStruct((2, N_GRAPHS, NODE_DIM), jnp.float32),
            jax.ShapeDtypeStruct((2, N_GRAPHS, NODE_DIM), jnp.float32),
        ],
        mesh=mesh,
        scratch_types=[
            pltpu.VMEM((L, NODE_DIM), jnp.float32),
            pltpu.VMEM((L, NODE_DIM), jnp.float32),
            pltpu.VMEM((40, L), jnp.int32),
            pltpu.VMEM((64, L), jnp.int32),
            pltpu.VMEM((N_TAIL, NODE_DIM), jnp.float32),
            pltpu.VMEM((N_TAIL,), jnp.int32),
            pltpu.VMEM((E_TAIL, NODE_DIM), jnp.float32),
            pltpu.VMEM((E_TAIL,), jnp.int32),
            pltpu.VMEM((NODE_DIM,), jnp.float32),
            pltpu.SemaphoreType.DMA,
            pltpu.SemaphoreType.DMA,
            pltpu.VMEM_SHARED((N_GRAPHS, NODE_DIM), jnp.float32),
            pltpu.VMEM_SHARED((N_GRAPHS + 1, NODE_DIM), jnp.float32),
        ],
    )
    np_part, ep_part = segsum(nodes, nid2d, ntail_id, epack, pid2d, ptail_id)

    w1a = lax.slice(W1, (0, 0), (NODE_DIM, HIDDEN))
    w1b = lax.slice(W1, (NODE_DIM, 0), (2 * NODE_DIM, HIDDEN))
    w1c = lax.slice(W1, (2 * NODE_DIM, 0), (2 * NODE_DIM + EDGE_DIM, HIDDEN))
    w1c_big = jnp.tile(w1c, (PACK, 1))

    out = pl.pallas_call(
        _mlp_body,
        out_shape=jax.ShapeDtypeStruct((N_GRAPHS, NODE_DIM), jnp.float32),
    )(np_part, ep_part, srows, sidsT, graph_globals, w1a, w1b, w1c_big,
      b1.reshape(1, HIDDEN), W2, b2.reshape(1, NODE_DIM))
    return out


# speedup vs baseline: 5.9857x; 1.1168x over previous
import jax
import jax.numpy as jnp
from jax import lax
from jax.experimental import pallas as pl
from jax.experimental.pallas import tpu as pltpu
from jax.experimental.pallas import tpu_sc as plsc

N_GRAPHS = 512
N_NODES = 100000
N_EDGES = 1600000
NODE_DIM = 128
EDGE_DIM = 16
HIDDEN = 64

NW = 32
L = 128

N_FULL = N_NODES // L
N_TAIL = N_NODES - N_FULL * L
N_CNT = N_FULL // NW
N_EXTRA = N_FULL - N_CNT * NW
N_MAX = N_CNT + 1

E_CHUNKS = N_EDGES // L
E_SLAB = 8
E_FULL = E_CHUNKS // E_SLAB
E_TCH = E_CHUNKS - E_FULL * E_SLAB
E_CNT = E_FULL // NW
E_EXTRA = E_FULL - E_CNT * NW
E_MAX = E_CNT + 1


def _node_phase(src_hbm, idxb, acc, buf0, buf1, sem0, sem1, start, cnt,
                idx_off):
    @pl.when(cnt > 0)
    def _():
        pltpu.async_copy(src_hbm.at[pl.ds(start * L, L), :], buf0, sem0)

    @pl.when(cnt > 1)
    def _():
        pltpu.async_copy(src_hbm.at[pl.ds((start + 1) * L, L), :], buf1, sem1)

    def _pair(tp, _):
        for half, (buf, sem) in enumerate(((buf0, sem0), (buf1, sem1))):
            t = 2 * tp + half

            @pl.when(t < cnt)
            def _():
                pltpu.make_async_copy(
                    src_hbm.at[pl.ds((start + t) * L, L), :], buf, sem).wait()
                pltpu.sync_copy(buf, acc.at[idxb.at[idx_off + t]], add=True)

                @pl.when(t + 2 < cnt)
                def _():
                    pltpu.async_copy(
                        src_hbm.at[pl.ds((start + t + 2) * L, L), :], buf, sem)
        return _

    lax.fori_loop(0, (N_MAX + 1) // 2, _pair, None)


def _edge_phase(src_hbm, idxb, acc, buf0, buf1, sem0, sem1, start, cnt):
    @pl.when(cnt > 0)
    def _():
        pltpu.async_copy(
            src_hbm.at[pl.ds(start * E_SLAB * L, E_SLAB * L), :], buf0, sem0)

    @pl.when(cnt > 1)
    def _():
        pltpu.async_copy(
            src_hbm.at[pl.ds((start + 1) * E_SLAB * L, E_SLAB * L), :],
            buf1, sem1)

    def _pair(tp, _):
        for half, (buf, sem) in enumerate(((buf0, sem0), (buf1, sem1))):
            t = 2 * tp + half

            @pl.when(t < cnt)
            def _():
                pltpu.make_async_copy(
                    src_hbm.at[pl.ds((start + t) * E_SLAB * L, E_SLAB * L), :],
                    buf, sem).wait()
                for j in range(E_SLAB):
                    pltpu.sync_copy(buf.at[pl.ds(j * L, L), :],
                                    acc.at[idxb.at[t * E_SLAB + j]], add=True)

                @pl.when(t + 2 < cnt)
                def _():
                    pltpu.async_copy(
                        src_hbm.at[pl.ds((start + t + 2) * E_SLAB * L,
                                         E_SLAB * L), :], buf, sem)
        return _

    lax.fori_loop(0, (E_MAX + 1) // 2, _pair, None)


def _segsum_body(nodes_hbm, edges_hbm, nid_hbm, ntail_id_hbm, eid_hbm,
                 np_out, ep_out,
                 nbuf0, nbuf1, ebuf0, ebuf1, nidxb, eidxb, ntrows, ntidx,
                 zrow, sem0, sem1, sem2, sem3, nacc, eacc):
    c = lax.axis_index("c")
    s = lax.axis_index("s")
    wid = c * 16 + s

    n_start = wid * N_CNT + jnp.minimum(wid, N_EXTRA)
    n_cnt = N_CNT + jnp.where(wid < N_EXTRA, 1, 0)
    e_start = wid * E_CNT + jnp.minimum(wid, E_EXTRA)
    e_cnt = E_CNT + jnp.where(wid < E_EXTRA, 1, 0)

    n_start8 = pl.multiple_of((n_start // 8) * 8, 8)
    pltpu.sync_copy(nid_hbm.at[pl.ds(n_start8, 40), :], nidxb)
    pltpu.sync_copy(eid_hbm.at[pl.ds(e_start * E_SLAB, E_MAX * E_SLAB), :],
                    eidxb)

    def _zero_row(i, _):
        zrow[pl.ds(i * 16, 16)] = jnp.zeros((16,), jnp.float32)
        return _
    lax.fori_loop(0, 8, _zero_row, None)
    base = s * (N_GRAPHS // 16)

    def _zero_nacc(i, _):
        pltpu.sync_copy(zrow, nacc.at[base + i])
        return _
    lax.fori_loop(0, N_GRAPHS // 16, _zero_nacc, None)

    def _zero_eacc(i, _):
        pltpu.sync_copy(zrow.at[pl.ds(0, EDGE_DIM)], eacc.at[base + i])
        return _
    lax.fori_loop(0, N_GRAPHS // 16, _zero_eacc, None)

    plsc.subcore_barrier()

    _node_phase(nodes_hbm, nidxb, nacc, nbuf0, nbuf1, sem0, sem1,
                n_start, n_cnt, n_start - n_start8)

    @pl.when(wid == 30)
    def _():
        pltpu.sync_copy(nodes_hbm.at[pl.ds(N_FULL * L, N_TAIL), :], ntrows)
        pltpu.sync_copy(ntail_id_hbm, ntidx)
        pltpu.sync_copy(ntrows, nacc.at[ntidx], add=True)

    _edge_phase(edges_hbm, eidxb, eacc, ebuf0, ebuf1, sem2, sem3,
                e_start, e_cnt)

    @pl.when(wid == NW - 1)
    def _():
        pltpu.sync_copy(edges_hbm.at[pl.ds(E_FULL * E_SLAB * L, E_TCH * L), :],
                        ebuf0.at[pl.ds(0, E_TCH * L), :])
        for j in range(E_TCH):
            pltpu.sync_copy(
                ebuf0.at[pl.ds(j * L, L), :],
                eacc.at[eidxb.at[(E_FULL - e_start * 1) * E_SLAB + j]],
                add=True)

    plsc.subcore_barrier()

    rows = N_GRAPHS // 16
    pltpu.sync_copy(nacc.at[pl.ds(s * rows, rows), :],
                    np_out.at[c, pl.ds(s * rows, rows), :])
    pltpu.sync_copy(eacc.at[pl.ds(s * rows, rows), :],
                    ep_out.at[c, pl.ds(s * rows, rows), :])


def _mlp_body(np_ref, ep_ref, gg_ref, w1a_ref, w1b_ref, w1c_ref, b1_ref,
              w2_ref, b2_ref, out_ref):
    ns = np_ref[0] + np_ref[1]
    es = ep_ref[0] + ep_ref[1]
    x = (jnp.dot(gg_ref[...], w1a_ref[...], preferred_element_type=jnp.float32)
         + jnp.dot(ns, w1b_ref[...], preferred_element_type=jnp.float32)
         + jnp.dot(es, w1c_ref[...], preferred_element_type=jnp.float32)
         + b1_ref[...])
    h = jnp.maximum(x, 0.0)
    out_ref[...] = (jnp.dot(h, w2_ref[...], preferred_element_type=jnp.float32)
                    + b2_ref[...])


def kernel(nodes, batch, edges, batch_edges, graph_globals, W1, b1, W2, b2):
    bid = batch.astype(jnp.int32)
    eid = batch_edges.astype(jnp.int32)
    nid2d = lax.slice(bid, (0,), (N_FULL * L,)).reshape(N_FULL, L)
    nid2d = jnp.pad(nid2d, ((0, N_MAX + NW), (0, 0)))
    ntail_id = lax.slice(bid, (N_FULL * L,), (N_NODES,))
    eid2d = eid.reshape(E_CHUNKS, L)
    eid2d = jnp.pad(eid2d, ((0, E_MAX * E_SLAB + NW), (0, 0)))

    mesh = plsc.VectorSubcoreMesh(core_axis_name="c", subcore_axis_name="s")
    segsum = pl.kernel(
        _segsum_body,
        out_type=[
            jax.ShapeDtypeStruct((2, N_GRAPHS, NODE_DIM), jnp.float32),
            jax.ShapeDtypeStruct((2, N_GRAPHS, EDGE_DIM), jnp.float32),
        ],
        mesh=mesh,
        compiler_params=pltpu.CompilerParams(use_tc_tiling_on_sc=False),
        scratch_types=[
            pltpu.VMEM((L, NODE_DIM), jnp.float32),
            pltpu.VMEM((L, NODE_DIM), jnp.float32),
            pltpu.VMEM((E_SLAB * L, EDGE_DIM), jnp.float32),
            pltpu.VMEM((E_SLAB * L, EDGE_DIM), jnp.float32),
            pltpu.VMEM((40, L), jnp.int32),
            pltpu.VMEM((E_MAX * E_SLAB, L), jnp.int32),
            pltpu.VMEM((N_TAIL, NODE_DIM), jnp.float32),
            pltpu.VMEM((N_TAIL,), jnp.int32),
            pltpu.VMEM((NODE_DIM,), jnp.float32),
            pltpu.SemaphoreType.DMA,
            pltpu.SemaphoreType.DMA,
            pltpu.SemaphoreType.DMA,
            pltpu.SemaphoreType.DMA,
            pltpu.VMEM_SHARED((N_GRAPHS, NODE_DIM), jnp.float32),
            pltpu.VMEM_SHARED((N_GRAPHS, EDGE_DIM), jnp.float32),
        ],
    )
    np_part, ep_part = segsum(nodes, edges, nid2d, ntail_id, eid2d)

    w1a = lax.slice(W1, (0, 0), (NODE_DIM, HIDDEN))
    w1b = lax.slice(W1, (NODE_DIM, 0), (2 * NODE_DIM, HIDDEN))
    w1c = lax.slice(W1, (2 * NODE_DIM, 0), (2 * NODE_DIM + EDGE_DIM, HIDDEN))

    out = pl.pallas_call(
        _mlp_body,
        out_shape=jax.ShapeDtypeStruct((N_GRAPHS, NODE_DIM), jnp.float32),
    )(np_part, ep_part, graph_globals, w1a, w1b, w1c,
      b1.reshape(1, HIDDEN), W2, b2.reshape(1, NODE_DIM))
    return out
